# baseline (speedup 1.0000x reference)
import functools

import jax
import jax.numpy as jnp
from jax import lax
from jax.experimental import pallas as pl
from jax.experimental.pallas import tpu as pltpu

SUBLANE = 8
ROTATING = 4


def _fused_kernel(x_hbm, gt_ref, b_ref, s1_ref, o_ref,
                  xbf, xrot, amx_ref, amn_ref, gst_ref, a_sems,
                  *, n_steps, tile_p, qn, qp, inverse):
    t = pl.program_id(0)
    c = gt_ref.shape[-1]

    def start_in(block, slot, sem):
        pltpu.make_async_copy(
            x_hbm.at[pl.ds(pl.multiple_of(block * tile_p, tile_p), tile_p), :],
            xrot.at[slot], sem).start()

    @pl.when(t == 0)
    def _():
        amx_ref[...] = jnp.full(amx_ref.shape, -jnp.inf, amx_ref.dtype)
        amn_ref[...] = jnp.full(amn_ref.shape, jnp.inf, amn_ref.dtype)
        for b in range(min(ROTATING, n_steps)):
            start_in(b, b, a_sems.at[b])

    @pl.when(t < n_steps)
    def _():
        slot = t & (ROTATING - 1)
        pltpu.make_async_copy(xrot.at[slot], xrot.at[slot],
                              a_sems.at[t]).wait()
        x = xrot[slot]
        x2 = (x * x).astype(jnp.bfloat16)
        xx = jnp.dot(x2, gt_ref[...], preferred_element_type=jnp.float32)
        xx3 = xx.reshape(tile_p // SUBLANE, SUBLANE, c)
        amx_ref[...] = jnp.maximum(amx_ref[...], jnp.max(xx3, axis=0))
        amn_ref[...] = jnp.minimum(amn_ref[...], jnp.min(xx3, axis=0))
        xbf[t] = x.astype(jnp.bfloat16)

        @pl.when(t + ROTATING < n_steps)
        def _():
            start_in(t + ROTATING, slot, a_sems.at[t + ROTATING])

    @pl.when(t == n_steps)
    def _():
        b = b_ref[...]
        gst_ref[0:1, :] = jnp.max(amx_ref[...], axis=0, keepdims=True) + b
        gst_ref[1:2, :] = jnp.min(amn_ref[...], axis=0, keepdims=True) + b

    @pl.when(t >= n_steps)
    def _():
        s = t - n_steps
        mx = gst_ref[0:1, :]
        mn = gst_ref[1:2, :]
        qscl = (mx - mn) * (0.9 / (qp - qn))
        qoff = mn * 0.9 - qn * qscl
        inv_qscl = 1.0 / qscl

        xb = xbf[s]
        x2 = xb * xb
        xx = jnp.dot(x2, gt_ref[...],
                     preferred_element_type=jnp.float32) + b_ref[...]

        x_hat = jnp.clip(jnp.round((xx - qoff) * inv_qscl), qn, qp)
        xq = x_hat * qscl + qoff

        if inverse:
            norm = jnp.sqrt(xq)
        else:
            norm = lax.rsqrt(xq)

        o_ref[...] = (s1_ref[...] * norm) * xb.astype(jnp.float32)


def _gdn_forward(x_nchw, gamma, beta, s1, *, num=256, inverse=False):
    N, C, H, W = x_nchw.shape
    P = N * H * W
    qn, qp = 0.0, float(num - 1)

    tile_p = 2048
    while P % tile_p and tile_p > SUBLANE:
        tile_p //= 2
    n_steps = P // tile_p

    x2d = jnp.transpose(x_nchw, (0, 2, 3, 1)).reshape(P, C)
    gt_bf = gamma.astype(jnp.bfloat16).T
    b_row = beta.astype(jnp.float32).reshape(1, C)
    s1_row = s1.astype(jnp.float32).reshape(1, C)

    out2d = pl.pallas_call(
        functools.partial(_fused_kernel, n_steps=n_steps, tile_p=tile_p,
                          qn=qn, qp=qp, inverse=inverse),
        out_shape=jax.ShapeDtypeStruct((P, C), jnp.float32),
        grid_spec=pltpu.PrefetchScalarGridSpec(
            num_scalar_prefetch=0,
            grid=(2 * n_steps,),
            in_specs=[
                pl.BlockSpec(memory_space=pl.ANY),
                pl.BlockSpec((C, C), lambda t: (0, 0)),
                pl.BlockSpec((1, C), lambda t: (0, 0)),
                pl.BlockSpec((1, C), lambda t: (0, 0)),
            ],
            out_specs=pl.BlockSpec(
                (tile_p, C),
                lambda t, _s=n_steps: (jnp.maximum(t - _s, 0), 0)),
            scratch_shapes=[
                pltpu.VMEM((n_steps, tile_p, C), jnp.bfloat16),
                pltpu.VMEM((ROTATING, tile_p, C), jnp.float32),
                pltpu.VMEM((SUBLANE, C), jnp.float32),
                pltpu.VMEM((SUBLANE, C), jnp.float32),
                pltpu.VMEM((2, C), jnp.float32),
                pltpu.SemaphoreType.DMA((n_steps,)),
            ],
        ),
        compiler_params=pltpu.CompilerParams(
            dimension_semantics=("arbitrary",),
            vmem_limit_bytes=54 * 1024 * 1024),
    )(x2d, gt_bf, b_row, s1_row)

    return out2d.reshape(N, H, W, C).transpose(0, 3, 1, 2)


def kernel(x, gamma, beta, s1):
    return _gdn_forward(x, gamma, beta, s1, num=256, inverse=False)


# speedup vs baseline: 1.1030x; 1.1030x over previous
import functools

import jax
import jax.numpy as jnp
from jax import lax
from jax.experimental import pallas as pl
from jax.experimental.pallas import tpu as pltpu

SUBLANE = 8
ROTATING = 2


def _fused_kernel(x_hbm, gt_ref, b_ref, s1_ref, o_ref,
                  xbf, xrot, amx_ref, amn_ref, gst_ref, a_sems,
                  *, n_steps, tile_p, qn, qp, inverse):
    t = pl.program_id(0)
    c = gt_ref.shape[-1]

    def start_in(block, slot, sem):
        pltpu.make_async_copy(
            x_hbm.at[pl.ds(pl.multiple_of(block * tile_p, tile_p), tile_p), :],
            xrot.at[slot], sem).start()

    @pl.when(t == 0)
    def _():
        amx_ref[...] = jnp.full(amx_ref.shape, -jnp.inf, amx_ref.dtype)
        amn_ref[...] = jnp.full(amn_ref.shape, jnp.inf, amn_ref.dtype)
        for b in range(min(ROTATING, n_steps)):
            start_in(b, b, a_sems.at[b])

    half = tile_p // 2

    @pl.when(t < n_steps)
    def _():
        slot = t & (ROTATING - 1)
        pltpu.make_async_copy(xrot.at[slot], xrot.at[slot],
                              a_sems.at[t]).wait()
        parts = []
        for h in range(2):
            x = xrot[slot, h * half:(h + 1) * half, :]
            x2 = (x * x).astype(jnp.bfloat16)
            xx = jnp.dot(x2, gt_ref[...], preferred_element_type=jnp.float32)
            xx3 = xx.reshape(half // SUBLANE, SUBLANE, c)
            parts.append((jnp.max(xx3, axis=0), jnp.min(xx3, axis=0)))
            xbf[t, h * half:(h + 1) * half, :] = x.astype(jnp.bfloat16)
        amx_ref[...] = jnp.maximum(amx_ref[...],
                                   jnp.maximum(parts[0][0], parts[1][0]))
        amn_ref[...] = jnp.minimum(amn_ref[...],
                                   jnp.minimum(parts[0][1], parts[1][1]))

        @pl.when(t + ROTATING < n_steps)
        def _():
            start_in(t + ROTATING, slot, a_sems.at[t + ROTATING])

    @pl.when(t == n_steps)
    def _():
        b = b_ref[...]
        gst_ref[0:1, :] = jnp.max(amx_ref[...], axis=0, keepdims=True) + b
        gst_ref[1:2, :] = jnp.min(amn_ref[...], axis=0, keepdims=True) + b

    @pl.when(t >= n_steps)
    def _():
        s = t - n_steps
        mx = gst_ref[0:1, :]
        mn = gst_ref[1:2, :]
        qscl = (mx - mn) * (0.9 / (qp - qn))
        qoff = mn * 0.9 - qn * qscl
        inv_qscl = 1.0 / qscl

        k1 = (b_ref[...] - qoff) * inv_qscl

        for h in range(2):
            xb = xbf[s, h * half:(h + 1) * half, :]
            x2 = xb * xb
            xxr = jnp.dot(x2, gt_ref[...],
                          preferred_element_type=jnp.float32)

            x_hat = jnp.clip(jnp.round(xxr * inv_qscl + k1), qn, qp)
            xq = x_hat * qscl + qoff

            if inverse:
                norm = jnp.sqrt(xq)
            else:
                norm = lax.rsqrt(xq)

            o_ref[h * half:(h + 1) * half, :] = (
                (s1_ref[...] * norm) * xb.astype(jnp.float32))


def _gdn_forward(x_nchw, gamma, beta, s1, *, num=256, inverse=False):
    N, C, H, W = x_nchw.shape
    P = N * H * W
    qn, qp = 0.0, float(num - 1)

    tile_p = 4096
    while P % tile_p and tile_p > SUBLANE:
        tile_p //= 2
    n_steps = P // tile_p

    x2d = jnp.transpose(x_nchw, (0, 2, 3, 1)).reshape(P, C)
    gt_bf = gamma.astype(jnp.bfloat16).T
    b_row = beta.astype(jnp.float32).reshape(1, C)
    s1_row = s1.astype(jnp.float32).reshape(1, C)

    out2d = pl.pallas_call(
        functools.partial(_fused_kernel, n_steps=n_steps, tile_p=tile_p,
                          qn=qn, qp=qp, inverse=inverse),
        out_shape=jax.ShapeDtypeStruct((P, C), jnp.float32),
        grid_spec=pltpu.PrefetchScalarGridSpec(
            num_scalar_prefetch=0,
            grid=(2 * n_steps,),
            in_specs=[
                pl.BlockSpec(memory_space=pl.ANY),
                pl.BlockSpec((C, C), lambda t: (0, 0)),
                pl.BlockSpec((1, C), lambda t: (0, 0)),
                pl.BlockSpec((1, C), lambda t: (0, 0)),
            ],
            out_specs=pl.BlockSpec(
                (tile_p, C),
                lambda t, _s=n_steps: (jnp.maximum(t - _s, 0), 0)),
            scratch_shapes=[
                pltpu.VMEM((n_steps, tile_p, C), jnp.bfloat16),
                pltpu.VMEM((ROTATING, tile_p, C), jnp.float32),
                pltpu.VMEM((SUBLANE, C), jnp.float32),
                pltpu.VMEM((SUBLANE, C), jnp.float32),
                pltpu.VMEM((2, C), jnp.float32),
                pltpu.SemaphoreType.DMA((n_steps,)),
            ],
        ),
        compiler_params=pltpu.CompilerParams(
            dimension_semantics=("arbitrary",),
            vmem_limit_bytes=54 * 1024 * 1024),
    )(x2d, gt_bf, b_row, s1_row)

    return out2d.reshape(N, H, W, C).transpose(0, 3, 1, 2)


def kernel(x, gamma, beta, s1):
    return _gdn_forward(x, gamma, beta, s1, num=256, inverse=False)
